# baseline (speedup 1.0000x reference)
import functools

import jax
import jax.numpy as jnp
from jax.experimental import pallas as pl

N = 10000
E = 160000
H = 256
HEADS = 8
DH = 32


def _proj_body(z_ref, wl_ref, wr_ref, xl_ref, xr_ref):
    z = z_ref[...]
    xl_ref[...] = jnp.dot(z, wl_ref[...], preferred_element_type=jnp.float32)
    xr_ref[...] = jnp.dot(z, wr_ref[...], preferred_element_type=jnp.float32)


def _proj(z, Wl, Wr):
    M = z.shape[0]
    BM = 1000
    return pl.pallas_call(
        _proj_body,
        grid=(M // BM,),
        in_specs=[
            pl.BlockSpec((BM, H), lambda i: (i, 0)),
            pl.BlockSpec((H, H), lambda i: (0, 0)),
            pl.BlockSpec((H, H), lambda i: (0, 0)),
        ],
        out_specs=[
            pl.BlockSpec((BM, H), lambda i: (i, 0)),
            pl.BlockSpec((BM, H), lambda i: (i, 0)),
        ],
        out_shape=[jax.ShapeDtypeStruct((M, H), jnp.float32)] * 2,
    )(z, Wl, Wr)


def _gatv2(z, src, dst, lp):
    xl2, xr2 = _proj(z, lp['Wl'], lp['Wr'])
    xl = xl2.reshape(-1, HEADS, DH)
    xr = xr2.reshape(-1, HEADS, DH)
    m = jax.nn.leaky_relu(xl[src] + xr[dst], 0.2)
    e = jnp.sum(m * lp['att'][None, :, :], axis=-1)
    emax = jax.ops.segment_max(e, dst, num_segments=z.shape[0])
    emax = jnp.where(jnp.isfinite(emax), emax, 0.0)
    ex = jnp.exp(e - emax[dst])
    den = jax.ops.segment_sum(ex, dst, num_segments=z.shape[0])
    alpha = ex / (den[dst] + 1e-16)
    out = jax.ops.segment_sum(alpha[:, :, None] * xl[src], dst,
                              num_segments=z.shape[0])
    return out.reshape(z.shape[0], HEADS * DH) + lp['b']


def kernel(x, edge_index, comps_first, comps_vectors, comps_third, expr_tree,
           params):
    src = edge_index[0].astype(jnp.int32)
    dst = edge_index[1].astype(jnp.int32)
    expr = jax.nn.relu(expr_tree @ params['We'] + params['be'])
    trans = jax.nn.relu(comps_vectors @ params['Wt'] + params['bt'])
    comp_features = jnp.concatenate([comps_first, trans, comps_third], axis=1)
    hc = jax.nn.relu(comp_features @ params['Wc1'] + params['bc1'])
    comp_emb = jax.nn.relu(hc @ params['Wc2'] + params['bc2'])
    z = x
    for lp in params['gat']:
        z = jax.nn.elu(_gatv2(z, src, dst, lp))
    graph_repr = jnp.mean(z, axis=0, keepdims=True)
    combined = jnp.concatenate(
        [graph_repr, comp_emb, jnp.mean(expr, axis=1)], axis=1)
    p1 = jax.nn.relu(combined @ params['Wp1'] + params['bp1'])
    p2 = jax.nn.relu(p1 @ params['Wp2'] + params['bp2'])
    out = p2 @ params['Wp3'] + params['bp3']
    return jax.nn.leaky_relu(jnp.squeeze(out, -1), 0.01)


# speedup vs baseline: 10.0595x; 10.0595x over previous
import functools

import jax
import jax.numpy as jnp
from jax import lax
from jax.experimental import pallas as pl
from jax.experimental.pallas import tpu as pltpu
from jax.experimental.pallas import tpu_sc as plsc

N = 10000
E = 160000
H = 256
HEADS = 8
DH = 32

N_PAD = 10240
E_PAD = 163840
NC = 2
NS = 16
NW = NC * NS
EPW = E_PAD // NW
BLK = 128
NBLK = EPW // BLK
RPS = N_PAD // NS
BM = 1024
QW = 64
NQ = H // QW


def _proj_x_body(x_ref, wl_ref, wr_ref, xl_ref, xr_ref):
    xz = x_ref[...]
    xl_ref[0] = jnp.dot(xz, wl_ref[...], preferred_element_type=jnp.float32)
    xr_ref[0] = jnp.dot(xz, wr_ref[...], preferred_element_type=jnp.float32)


def _proj_x(x_pad, Wl, Wr):
    return pl.pallas_call(
        _proj_x_body,
        grid=(N_PAD // BM, 2),
        in_specs=[
            pl.BlockSpec((BM, H), lambda i, h: (i, 0)),
            pl.BlockSpec((H, H // 2), lambda i, h: (0, h)),
            pl.BlockSpec((H, H // 2), lambda i, h: (0, h)),
        ],
        out_specs=[
            pl.BlockSpec((1, BM, H // 2), lambda i, h: (h, i, 0)),
            pl.BlockSpec((1, BM, H // 2), lambda i, h: (h, i, 0)),
        ],
        out_shape=[jax.ShapeDtypeStruct((2, N_PAD, H // 2), jnp.float32)] * 2,
    )(x_pad, Wl, Wr)


def _elu(v):
    return jnp.where(v > 0, v, jnp.exp(v) - jnp.float32(1.0))


def _norm_z(p0_ref, p1_ref, d0_ref, d1_ref, sel_ref, b_ref):
    g = p0_ref[...] + p1_ref[...]
    recip = jnp.float32(1.0) / (d0_ref[...] + d1_ref[...] + jnp.float32(1e-16))
    divrep = jnp.dot(recip, sel_ref[...], preferred_element_type=jnp.float32)
    return _elu(g * divrep + b_ref[...])


def _proj_parts_body(p0_ref, p1_ref, d0_ref, d1_ref, sel_ref, b_ref,
                     wl_ref, wr_ref, xl_ref, xr_ref):
    zc = _norm_z(p0_ref, p1_ref, d0_ref, d1_ref, sel_ref, b_ref)
    xl_ref[0] = jnp.dot(zc, wl_ref[...], preferred_element_type=jnp.float32)
    xr_ref[0] = jnp.dot(zc, wr_ref[...], preferred_element_type=jnp.float32)


def _proj_parts(o0, o1, d0, d1, sel, b, Wl, Wr):
    return pl.pallas_call(
        _proj_parts_body,
        grid=(N_PAD // BM, 2),
        in_specs=[
            pl.BlockSpec((BM, H), lambda i, h: (i, 0)),
            pl.BlockSpec((BM, H), lambda i, h: (i, 0)),
            pl.BlockSpec((BM, 16), lambda i, h: (i, 0)),
            pl.BlockSpec((BM, 16), lambda i, h: (i, 0)),
            pl.BlockSpec((16, H), lambda i, h: (0, 0)),
            pl.BlockSpec((1, H), lambda i, h: (0, 0)),
            pl.BlockSpec((H, H // 2), lambda i, h: (0, h)),
            pl.BlockSpec((H, H // 2), lambda i, h: (0, h)),
        ],
        out_specs=[
            pl.BlockSpec((1, BM, H // 2), lambda i, h: (h, i, 0)),
            pl.BlockSpec((1, BM, H // 2), lambda i, h: (h, i, 0)),
        ],
        out_shape=[jax.ShapeDtypeStruct((2, N_PAD, H // 2), jnp.float32)] * 2,
    )(o0, o1, d0, d1, sel, b, Wl, Wr)


def _combine_body(p0_ref, p1_ref, d0_ref, d1_ref, sel_ref, b_ref, z_ref):
    z_ref[...] = _norm_z(p0_ref, p1_ref, d0_ref, d1_ref, sel_ref, b_ref)


def _combine(o0, o1, d0, d1, sel, b):
    return pl.pallas_call(
        _combine_body,
        grid=(N_PAD // BM,),
        in_specs=[
            pl.BlockSpec((BM, H), lambda i: (i, 0)),
            pl.BlockSpec((BM, H), lambda i: (i, 0)),
            pl.BlockSpec((BM, 16), lambda i: (i, 0)),
            pl.BlockSpec((BM, 16), lambda i: (i, 0)),
            pl.BlockSpec((16, H), lambda i: (0, 0)),
            pl.BlockSpec((1, H), lambda i: (0, 0)),
        ],
        out_specs=pl.BlockSpec((BM, H), lambda i: (i, 0)),
        out_shape=jax.ShapeDtypeStruct((N_PAD, H), jnp.float32),
    )(o0, o1, d0, d1, sel, b)


_MESH = plsc.VectorSubcoreMesh(core_axis_name="c", subcore_axis_name="s")

_GDN = lax.GatherDimensionNumbers(offset_dims=(), collapsed_slice_dims=(0,),
                                  start_index_map=(0,))


def _lane_gather(v, idx):
    return lax.gather(v, idx[:, None], _GDN, (1,),
                      mode=lax.GatherScatterMode.PROMISE_IN_BOUNDS)


@functools.partial(
    pl.kernel,
    mesh=_MESH,
    out_type=[
        jax.ShapeDtypeStruct((E_PAD, 16), jnp.float32),
        jax.ShapeDtypeStruct((N_PAD, 16), jnp.float32),
        jax.ShapeDtypeStruct((N_PAD, 16), jnp.float32),
    ],
    scratch_types=[
        pltpu.VMEM((BLK,), jnp.int32),
        pltpu.VMEM((BLK,), jnp.int32),
        pltpu.VMEM((BLK, H // 2), jnp.float32),
        pltpu.VMEM((BLK, H // 2), jnp.float32),
        pltpu.VMEM((BLK, H // 2), jnp.float32),
        pltpu.VMEM((BLK, H // 2), jnp.float32),
        pltpu.VMEM((BLK, 16), jnp.float32),
        pltpu.VMEM((H,), jnp.float32),
        pltpu.VMEM((BLK, 16), jnp.float32),
        pltpu.VMEM((BLK,), jnp.int32),
        pltpu.VMEM_SHARED((N_PAD, 16), jnp.float32),
        pltpu.SemaphoreType.DMA,
    ],
)
def _sc_pass1(xl0_hbm, xl1_hbm, xr0_hbm, xr1_hbm, att_hbm, src_hbm, dst_hbm,
              ex_hbm, den0_hbm, den1_hbm,
              sidx, didx, xl0b, xl1b, xr0b, xr1b, exb, attb, zb, ridx, den_sp,
              sem):
    c = lax.axis_index("c")
    s = lax.axis_index("s")
    wid = s * NC + c
    pltpu.sync_copy(att_hbm, attb)
    zv = jnp.zeros((16,), jnp.float32)

    def zrow(i, carry):
        zb[i, :] = zv
        return carry

    lax.fori_loop(0, BLK, zrow, 0)
    lanes = lax.iota(jnp.int32, 16)

    def fill_ridx(base):
        for k in range(BLK // 16):
            ridx[pl.ds(16 * k, 16)] = base + 16 * k + lanes

    for i in range(RPS // BLK):
        fill_ridx(s * RPS + i * BLK)
        pltpu.sync_copy(zb, den_sp.at[ridx])
    plsc.subcore_barrier()

    perms = [lanes ^ sh for sh in (8, 4, 2, 1)]
    att_regs = [attb[pl.ds(16 * i, 16)] for i in range(16)]

    def blk_body(blk, carry):
        base = wid * EPW + blk * BLK
        pltpu.sync_copy(src_hbm.at[pl.ds(base, BLK)], sidx)
        pltpu.sync_copy(dst_hbm.at[pl.ds(base, BLK)], didx)
        cps = [
            pltpu.async_copy(xl0_hbm.at[sidx], xl0b, sem),
            pltpu.async_copy(xl1_hbm.at[sidx], xl1b, sem),
            pltpu.async_copy(xr0_hbm.at[didx], xr0b, sem),
            pltpu.async_copy(xr1_hbm.at[didx], xr1b, sem),
        ]
        for cp in cps:
            cp.wait()

        def edge_body(b, ecarry):
            evec = jnp.zeros((16,), jnp.float32)
            for h in range(HEADS):
                th = None
                for j in range(2):
                    i = 2 * h + j
                    q, off = divmod(16 * i, H // 2)
                    xlb_ = (xl0b, xl1b)[q]
                    xrb_ = (xr0b, xr1b)[q]
                    sv = (xlb_[b, pl.ds(off, 16)]
                          + xrb_[b, pl.ds(off, 16)])
                    m = jnp.maximum(sv, sv * jnp.float32(0.2))
                    t = m * att_regs[i]
                    th = t if th is None else th + t
                for p in perms:
                    th = th + _lane_gather(th, p)
                evec = jnp.where(lanes == h, th, evec)
            exv = jnp.where(lanes < HEADS, jnp.exp(evec), jnp.float32(0.0))
            exb[b, :] = exv
            return ecarry

        lax.fori_loop(0, BLK, edge_body, 0)
        pltpu.sync_copy(exb, ex_hbm.at[pl.ds(base, BLK)])
        pltpu.sync_copy(exb, den_sp.at[didx], add=True)
        return carry

    lax.fori_loop(0, NBLK, blk_body, 0)
    plsc.subcore_barrier()

    def cpout(i, carry):
        r = s * RPS + i * BLK
        fill_ridx(r)
        pltpu.async_copy(den_sp.at[ridx], zb, sem).wait()

        @pl.when(c == 0)
        def _():
            pltpu.sync_copy(zb, den0_hbm.at[pl.ds(r, BLK)])

        @pl.when(c == 1)
        def _():
            pltpu.sync_copy(zb, den1_hbm.at[pl.ds(r, BLK)])

        return carry

    lax.fori_loop(0, RPS // BLK, cpout, 0)


@functools.partial(
    pl.kernel,
    mesh=_MESH,
    out_type=[
        jax.ShapeDtypeStruct((NQ, N_PAD, QW), jnp.float32),
        jax.ShapeDtypeStruct((NQ, N_PAD, QW), jnp.float32),
    ],
    scratch_types=[
        pltpu.VMEM((BLK,), jnp.int32),
        pltpu.VMEM((BLK,), jnp.int32),
        pltpu.VMEM((BLK, H // 2), jnp.float32),
        pltpu.VMEM((BLK, QW), jnp.float32),
        pltpu.VMEM((BLK, 16), jnp.float32),
        pltpu.VMEM((BLK, QW), jnp.float32),
        pltpu.VMEM((BLK,), jnp.int32),
        pltpu.VMEM_SHARED((N_PAD, QW), jnp.float32),
        pltpu.SemaphoreType.DMA,
    ],
)
def _sc_pass2(xl0_hbm, xl1_hbm, ex_hbm, src_hbm, dst_hbm,
              out0_hbm, out1_hbm,
              sidx, didx, xlb, msgb, exb, zb, ridx, out_sp, sem):
    c = lax.axis_index("c")
    s = lax.axis_index("s")
    wid = s * NC + c
    lanes = lax.iota(jnp.int32, 16)
    zv = jnp.zeros((16,), jnp.float32)

    def zrow(i, carry):
        for k in range(QW // 16):
            zb[i, pl.ds(16 * k, 16)] = zv
        return carry

    lax.fori_loop(0, BLK, zrow, 0)

    def fill_ridx(base):
        for k in range(BLK // 16):
            ridx[pl.ds(16 * k, 16)] = base + 16 * k + lanes

    for q in range(NQ):
        for i in range(RPS // BLK):
            fill_ridx(s * RPS + i * BLK)
            pltpu.sync_copy(zb, out_sp.at[ridx])
        plsc.subcore_barrier()

        def blk_body(blk, carry):
            base = wid * EPW + blk * BLK
            pltpu.sync_copy(src_hbm.at[pl.ds(base, BLK)], sidx)
            pltpu.sync_copy(dst_hbm.at[pl.ds(base, BLK)], didx)
            xl_hbm = (xl0_hbm, xl1_hbm)[q // 2]
            cp = pltpu.async_copy(xl_hbm.at[sidx], xlb, sem)
            pltpu.sync_copy(ex_hbm.at[pl.ds(base, BLK)], exb)
            cp.wait()

            def edge_body(b, ecarry):
                exrow = exb[b, :]
                for hh in range(2):
                    head = 2 * q + hh
                    a_bc = _lane_gather(exrow,
                                        jnp.full((16,), head, jnp.int32))
                    for j in range(2):
                        off = 32 * hh + 16 * j
                        src_off = (q % 2) * QW + off
                        msgb[b, pl.ds(off, 16)] = (
                            a_bc * xlb[b, pl.ds(src_off, 16)])
                return ecarry

            lax.fori_loop(0, BLK, edge_body, 0)
            pltpu.sync_copy(msgb, out_sp.at[didx], add=True)
            return carry

        lax.fori_loop(0, NBLK, blk_body, 0)
        plsc.subcore_barrier()

        def cpout(i, carry):
            r = s * RPS + i * BLK
            fill_ridx(r)
            pltpu.async_copy(out_sp.at[ridx], zb, sem).wait()

            @pl.when(c == 0)
            def _():
                pltpu.sync_copy(zb, out0_hbm.at[q].at[pl.ds(r, BLK)])

            @pl.when(c == 1)
            def _():
                pltpu.sync_copy(zb, out1_hbm.at[q].at[pl.ds(r, BLK)])

            return carry

        lax.fori_loop(0, RPS // BLK, cpout, 0)
        plsc.subcore_barrier()


def kernel(x, edge_index, comps_first, comps_vectors, comps_third, expr_tree,
           params):
    src = edge_index[0].astype(jnp.int32)
    dst = edge_index[1].astype(jnp.int32)
    srcp = jnp.concatenate([src, jnp.zeros((E_PAD - E,), jnp.int32)])
    dstp = jnp.concatenate([dst, jnp.full((E_PAD - E,), N, jnp.int32)])

    expr = jax.nn.relu(expr_tree @ params['We'] + params['be'])
    trans = jax.nn.relu(comps_vectors @ params['Wt'] + params['bt'])
    comp_features = jnp.concatenate([comps_first, trans, comps_third], axis=1)
    hc = jax.nn.relu(comp_features @ params['Wc1'] + params['bc1'])
    comp_emb = jax.nn.relu(hc @ params['Wc2'] + params['bc2'])

    gat = params['gat']
    x_pad = jnp.pad(x, ((0, N_PAD - N), (0, 0)))
    sel = (jnp.arange(H, dtype=jnp.int32)[None, :] // DH
           == jnp.arange(16, dtype=jnp.int32)[:, None]).astype(jnp.float32)
    xl, xr = _proj_x(x_pad, gat[0]['Wl'], gat[0]['Wr'])
    z = None
    for l in range(len(gat)):
        lp = gat[l]
        attv = lp['att'].reshape(-1)
        ex, den0, den1 = _sc_pass1(xl[0], xl[1], xr[0], xr[1], attv, srcp,
                                   dstp)
        o0q, o1q = _sc_pass2(xl[0], xl[1], ex, srcp, dstp)
        o0 = o0q.transpose(1, 0, 2).reshape(N_PAD, H)
        o1 = o1q.transpose(1, 0, 2).reshape(N_PAD, H)
        b = lp['b'].reshape(1, -1)
        if l + 1 < len(gat):
            xl, xr = _proj_parts(o0, o1, den0, den1, sel, b,
                                 gat[l + 1]['Wl'], gat[l + 1]['Wr'])
        else:
            z = _combine(o0, o1, den0, den1, sel, b)

    graph_repr = jnp.mean(z[:N], axis=0, keepdims=True)
    combined = jnp.concatenate(
        [graph_repr, comp_emb, jnp.mean(expr, axis=1)], axis=1)
    p1 = jax.nn.relu(combined @ params['Wp1'] + params['bp1'])
    p2 = jax.nn.relu(p1 @ params['Wp2'] + params['bp2'])
    out = p2 @ params['Wp3'] + params['bp3']
    return jax.nn.leaky_relu(jnp.squeeze(out, -1), 0.01)
